# baseline (speedup 1.0000x reference)
import jax
import jax.numpy as jnp
from jax.experimental import pallas as pl
from jax.experimental.pallas import tpu as pltpu

B = 32
H = 200
S = 30
A = 6
PLAN = 12
ITERS = 3
CAND = 1000
TOPK = 100

SEG = 1024
G = 8
NB = B // G
W = G * SEG

_INT32_MIN = -2147483648


def _body(eps_ref, h0_ref, s0_ref, WhhT_ref, WahT_ref, WssT_ref, WhsT_ref,
          wrh_ref, wrs_ref, out_ref, ret_scr, mean_scr, std_scr):
    it = pl.program_id(0)
    j = pl.program_id(1)

    @pl.when((it == 0) & (j == 0))
    def _init():
        mean_scr[...] = jnp.zeros((B, PLAN, A), jnp.float32)
        std_scr[...] = jnp.ones((B, PLAN, A), jnp.float32)

    def bdot(x, w):
        return jnp.dot(x, w, preferred_element_type=jnp.float32)

    @pl.when(j < NB)
    def _rollout():
        WhhT = WhhT_ref[...]
        WahT = WahT_ref[...]
        WssT = WssT_ref[...]
        WhsT = WhsT_ref[...]
        wrh = wrh_ref[...]
        wrs = wrs_ref[...]

        bidx = [g * NB + j for g in range(G)]
        mean_jT = [jnp.transpose(mean_scr[pl.ds(b, 1)].reshape(PLAN, A))
                   for b in bidx]
        std_jT = [jnp.transpose(std_scr[pl.ds(b, 1)].reshape(PLAN, A))
                  for b in bidx]
        h0 = [jnp.transpose(h0_ref[pl.ds(b, 1)].reshape(1, H))
              .astype(jnp.bfloat16) for b in bidx]
        s0 = [jnp.transpose(s0_ref[pl.ds(b, 1)].reshape(1, S))
              .astype(jnp.bfloat16) for b in bidx]

        def seg_bcast(cols, n):
            return jnp.concatenate(
                [jnp.broadcast_to(c, (n, SEG)) for c in cols], axis=1)

        ret = jnp.zeros((1, W), jnp.float32)
        h = None
        s = None
        for p in range(PLAN):
            m_full = seg_bcast([m[:, p:p + 1] for m in mean_jT], A)
            sd_full = seg_bcast([sd[:, p:p + 1] for sd in std_jT], A)
            a = (m_full + sd_full * eps_ref[0, j, p]).astype(jnp.bfloat16)
            if p == 0:
                hh = seg_bcast([bdot(WhhT, c) for c in h0], H)
                ss = seg_bcast([bdot(WssT, c) for c in s0], S)
            else:
                hh = bdot(WhhT, h)
                ss = bdot(WssT, s)
            h = jnp.tanh(hh + bdot(WahT, a)).astype(jnp.bfloat16)
            s = jnp.tanh(ss + bdot(WhsT, h)).astype(jnp.bfloat16)
            ret = ret + bdot(wrh, h) + bdot(wrs, s)
        ret_scr[pl.ds(j, 1)] = ret.reshape(1, 1, W)

    @pl.when(j == NB)
    def _select():
        ret = ret_scr[...].reshape(NB, W)
        bits = jax.lax.bitcast_convert_type(ret, jnp.int32)
        int_min = jnp.int32(_INT32_MIN)
        skey = jnp.where(bits >= 0, bits,
                         jnp.bitwise_xor(jnp.invert(bits), int_min))
        lane = jax.lax.broadcasted_iota(jnp.int32, (NB, W), 1)
        skey = jnp.where((lane & (SEG - 1)) >= CAND, int_min, skey)

        epsb = eps_ref[0]
        for g in range(G):
            sk = skey[:, g * SEG:(g + 1) * SEG]

            def count_ge(t):
                return jnp.sum((sk >= t).astype(jnp.int32), axis=1,
                               keepdims=True)

            zero = jnp.zeros((NB, 1), jnp.int32)
            t = jnp.where(count_ge(zero) >= TOPK, zero,
                          jnp.full((NB, 1), _INT32_MIN, jnp.int32))
            for bit in range(30, -1, -1):
                cand_t = t + jnp.int32(1 << bit)
                t = jnp.where(count_ge(cand_t) >= TOPK, cand_t, t)
            mask = sk >= t
            cnt = jnp.sum(mask.astype(jnp.float32), axis=1, keepdims=True)
            inv = (1.0 / cnt).reshape(NB, 1, 1)

            seg = epsb[:, :, :, g * SEG:(g + 1) * SEG]
            esel = jnp.where(mask[:, None, None, :], seg, 0.0)
            s1 = jnp.sum(esel, axis=3)
            s2 = jnp.sum(esel * esel, axis=3)
            mu = s1 * inv
            var = s2 * inv - mu * mu
            sd = jnp.sqrt(jnp.maximum(var, 0.0))
            rows = slice(g * NB, (g + 1) * NB)
            old_std = std_scr[rows]
            mean_scr[rows] = mean_scr[rows] + old_std * mu
            std_scr[rows] = old_std * sd

        @pl.when(it == ITERS - 1)
        def _out():
            out_ref[...] = mean_scr[...][:, 0, :]


@jax.jit
def kernel(hidden, state, W_hh, W_ah, W_ss, W_hs, w_rh, w_rs):
    base = jax.random.key(42)
    eps = jnp.stack([
        jax.random.normal(jax.random.fold_in(base, it), (PLAN, B, CAND, A),
                          dtype=hidden.dtype)
        for it in range(ITERS)
    ])
    epsT = jnp.transpose(eps, (0, 2, 1, 4, 3))
    epsT = jnp.pad(epsT, ((0, 0), (0, 0), (0, 0), (0, 0), (0, SEG - CAND)))
    epsT = epsT.reshape(ITERS, G, NB, PLAN, A, SEG)
    epsT = jnp.transpose(epsT, (0, 2, 3, 4, 1, 5)).reshape(
        ITERS, NB, PLAN, A, W)

    grid = (ITERS, NB + 1)
    out = pl.pallas_call(
        _body,
        grid=grid,
        in_specs=[
            pl.BlockSpec((1, NB, PLAN, A, W), lambda it, j: (it, 0, 0, 0, 0)),
            pl.BlockSpec((B, 1, H), lambda it, j: (0, 0, 0)),
            pl.BlockSpec((B, 1, S), lambda it, j: (0, 0, 0)),
            pl.BlockSpec((H, H), lambda it, j: (0, 0)),
            pl.BlockSpec((H, A), lambda it, j: (0, 0)),
            pl.BlockSpec((S, S), lambda it, j: (0, 0)),
            pl.BlockSpec((S, H), lambda it, j: (0, 0)),
            pl.BlockSpec((1, H), lambda it, j: (0, 0)),
            pl.BlockSpec((1, S), lambda it, j: (0, 0)),
        ],
        out_specs=pl.BlockSpec((B, A), lambda it, j: (0, 0)),
        out_shape=jax.ShapeDtypeStruct((B, A), jnp.float32),
        scratch_shapes=[
            pltpu.VMEM((NB, 1, W), jnp.float32),
            pltpu.VMEM((B, PLAN, A), jnp.float32),
            pltpu.VMEM((B, PLAN, A), jnp.float32),
        ],
    )(epsT, hidden.reshape(B, 1, H), state.reshape(B, 1, S),
      W_hh.T.astype(jnp.bfloat16), W_ah.T.astype(jnp.bfloat16),
      W_ss.T.astype(jnp.bfloat16), W_hs.T.astype(jnp.bfloat16),
      w_rh.reshape(1, H).astype(jnp.bfloat16),
      w_rs.reshape(1, S).astype(jnp.bfloat16))
    return out


# speedup vs baseline: 1.3388x; 1.3388x over previous
import jax
import jax.numpy as jnp
from jax.experimental import pallas as pl
from jax.experimental.pallas import tpu as pltpu

B = 32
H = 200
S = 30
A = 6
PLAN = 12
ITERS = 3
CAND = 1000
TOPK = 100

SEG = 1024
G = 4
NB = B // G
W = G * SEG
CH = 2

_INT32_MIN = -2147483648


def _body(eps_ref, h0_ref, s0_ref, WhhT_ref, WahT_ref, WssT_ref, WhsT_ref,
          wrh_ref, wrs_ref, out_ref, ret_scr, mean_scr, std_scr):
    it = pl.program_id(0)
    j = pl.program_id(1)

    @pl.when((it == 0) & (j == 0))
    def _init():
        mean_scr[...] = jnp.zeros((B, PLAN, A), jnp.float32)
        std_scr[...] = jnp.ones((B, PLAN, A), jnp.float32)

    def bdot(x, w):
        return jnp.dot(x, w, preferred_element_type=jnp.float32)

    @pl.when(j < NB)
    def _rollout():
        WhhT = WhhT_ref[...]
        WahT = WahT_ref[...]
        WssT = WssT_ref[...]
        WhsT = WhsT_ref[...]
        wrh = wrh_ref[...]
        wrs = wrs_ref[...]

        bidx = [g * NB + j for g in range(G)]
        mean_jT = [jnp.transpose(mean_scr[pl.ds(b, 1)].reshape(PLAN, A))
                   for b in bidx]
        std_jT = [jnp.transpose(std_scr[pl.ds(b, 1)].reshape(PLAN, A))
                  for b in bidx]
        h0 = [jnp.transpose(h0_ref[pl.ds(b, 1)].reshape(1, H))
              .astype(jnp.bfloat16) for b in bidx]
        s0 = [jnp.transpose(s0_ref[pl.ds(b, 1)].reshape(1, S))
              .astype(jnp.bfloat16) for b in bidx]

        def seg_bcast(cols, n):
            if len(cols) == 1:
                return jnp.broadcast_to(cols[0], (n, SEG))
            return jnp.concatenate(
                [jnp.broadcast_to(c, (n, SEG)) for c in cols], axis=1)

        GC = G // CH
        WC = GC * SEG
        cseg = [slice(c * WC, (c + 1) * WC) for c in range(CH)]
        cgrp = [slice(c * GC, (c + 1) * GC) for c in range(CH)]
        ret = [jnp.zeros((1, WC), jnp.float32) for _ in range(CH)]
        h = [None] * CH
        s = [None] * CH
        for p in range(PLAN):
            eps_p = eps_ref[0, j, p]
            a = [None] * CH
            for c in range(CH):
                m_full = seg_bcast([m[:, p:p + 1]
                                    for m in mean_jT[cgrp[c]]], A)
                sd_full = seg_bcast([sd[:, p:p + 1]
                                     for sd in std_jT[cgrp[c]]], A)
                a[c] = (m_full +
                        sd_full * eps_p[:, cseg[c]]).astype(jnp.bfloat16)
            for c in range(CH):
                if p == 0:
                    hh = seg_bcast([bdot(WhhT, col) for col in h0[cgrp[c]]], H)
                    ss = seg_bcast([bdot(WssT, col) for col in s0[cgrp[c]]], S)
                else:
                    hh = bdot(WhhT, h[c])
                    ss = bdot(WssT, s[c])
                h[c] = jnp.tanh(hh + bdot(WahT, a[c])).astype(jnp.bfloat16)
                s[c] = jnp.tanh(ss + bdot(WhsT, h[c])).astype(jnp.bfloat16)
                ret[c] = ret[c] + bdot(wrh, h[c]) + bdot(wrs, s[c])
        ret_full = ret[0] if CH == 1 else jnp.concatenate(ret, axis=1)
        ret_scr[pl.ds(j, 1)] = ret_full.reshape(1, 1, W)

    @pl.when(j == NB)
    def _select():
        ret = ret_scr[...].reshape(NB, W)
        bits = jax.lax.bitcast_convert_type(ret, jnp.int32)
        int_min = jnp.int32(_INT32_MIN)
        skey = jnp.where(bits >= 0, bits,
                         jnp.bitwise_xor(jnp.invert(bits), int_min))
        lane = jax.lax.broadcasted_iota(jnp.int32, (NB, W), 1)
        skey = jnp.where((lane & (SEG - 1)) >= CAND, int_min, skey)

        epsb = eps_ref[0]
        for g in range(G):
            sk = skey[:, g * SEG:(g + 1) * SEG]

            def count_ge(t):
                return jnp.sum((sk >= t).astype(jnp.int32), axis=1,
                               keepdims=True)

            zero = jnp.zeros((NB, 1), jnp.int32)
            t = jnp.where(count_ge(zero) >= TOPK, zero,
                          jnp.full((NB, 1), _INT32_MIN, jnp.int32))
            for bit in range(30, -1, -1):
                cand_t = t + jnp.int32(1 << bit)
                t = jnp.where(count_ge(cand_t) >= TOPK, cand_t, t)
            mask = sk >= t
            cnt = jnp.sum(mask.astype(jnp.float32), axis=1, keepdims=True)
            inv = (1.0 / cnt).reshape(NB, 1, 1)

            seg = epsb[:, :, :, g * SEG:(g + 1) * SEG]
            esel = jnp.where(mask[:, None, None, :], seg, 0.0)
            s1 = jnp.sum(esel, axis=3)
            s2 = jnp.sum(esel * esel, axis=3)
            mu = s1 * inv
            var = s2 * inv - mu * mu
            sd = jnp.sqrt(jnp.maximum(var, 0.0))
            rows = slice(g * NB, (g + 1) * NB)
            old_std = std_scr[rows]
            mean_scr[rows] = mean_scr[rows] + old_std * mu
            std_scr[rows] = old_std * sd

        @pl.when(it == ITERS - 1)
        def _out():
            out_ref[...] = mean_scr[...][:, 0, :]


@jax.jit
def kernel(hidden, state, W_hh, W_ah, W_ss, W_hs, w_rh, w_rs):
    base = jax.random.key(42)
    eps = jnp.stack([
        jax.random.normal(jax.random.fold_in(base, it), (PLAN, B, CAND, A),
                          dtype=hidden.dtype)
        for it in range(ITERS)
    ])
    epsT = jnp.transpose(eps, (0, 2, 1, 4, 3))
    epsT = jnp.pad(epsT, ((0, 0), (0, 0), (0, 0), (0, 0), (0, SEG - CAND)))
    epsT = epsT.reshape(ITERS, G, NB, PLAN, A, SEG)
    epsT = jnp.transpose(epsT, (0, 2, 3, 4, 1, 5)).reshape(
        ITERS, NB, PLAN, A, W)

    grid = (ITERS, NB + 1)
    out = pl.pallas_call(
        _body,
        grid=grid,
        in_specs=[
            pl.BlockSpec((1, NB, PLAN, A, W), lambda it, j: (it, 0, 0, 0, 0)),
            pl.BlockSpec((B, 1, H), lambda it, j: (0, 0, 0)),
            pl.BlockSpec((B, 1, S), lambda it, j: (0, 0, 0)),
            pl.BlockSpec((H, H), lambda it, j: (0, 0)),
            pl.BlockSpec((H, A), lambda it, j: (0, 0)),
            pl.BlockSpec((S, S), lambda it, j: (0, 0)),
            pl.BlockSpec((S, H), lambda it, j: (0, 0)),
            pl.BlockSpec((1, H), lambda it, j: (0, 0)),
            pl.BlockSpec((1, S), lambda it, j: (0, 0)),
        ],
        out_specs=pl.BlockSpec((B, A), lambda it, j: (0, 0)),
        out_shape=jax.ShapeDtypeStruct((B, A), jnp.float32),
        scratch_shapes=[
            pltpu.VMEM((NB, 1, W), jnp.float32),
            pltpu.VMEM((B, PLAN, A), jnp.float32),
            pltpu.VMEM((B, PLAN, A), jnp.float32),
        ],
    )(epsT, hidden.reshape(B, 1, H), state.reshape(B, 1, S),
      W_hh.T.astype(jnp.bfloat16), W_ah.T.astype(jnp.bfloat16),
      W_ss.T.astype(jnp.bfloat16), W_hs.T.astype(jnp.bfloat16),
      w_rh.reshape(1, H).astype(jnp.bfloat16),
      w_rs.reshape(1, S).astype(jnp.bfloat16))
    return out


# speedup vs baseline: 1.3558x; 1.0127x over previous
import jax
import jax.numpy as jnp
from jax.experimental import pallas as pl
from jax.experimental.pallas import tpu as pltpu

B = 32
H = 200
S = 30
A = 6
PLAN = 12
ITERS = 3
CAND = 1000
TOPK = 100

SEG = 1024
G = 4
NB = B // G
W = G * SEG
CH = 1

_INT32_MIN = -2147483648


def _body(eps_ref, h0_ref, s0_ref, WhhT_ref, WahT_ref, WssT_ref, WhsT_ref,
          wrh_ref, wrs_ref, out_ref, ret_scr, mean_scr, std_scr):
    it = pl.program_id(0)
    j = pl.program_id(1)

    @pl.when((it == 0) & (j == 0))
    def _init():
        mean_scr[...] = jnp.zeros((B, PLAN, A), jnp.float32)
        std_scr[...] = jnp.ones((B, PLAN, A), jnp.float32)

    def bdot(x, w):
        return jnp.dot(x, w, preferred_element_type=jnp.float32)

    @pl.when(j < NB)
    def _rollout():
        WhhT = WhhT_ref[...]
        WahT = WahT_ref[...]
        WssT = WssT_ref[...]
        WhsT = WhsT_ref[...]
        wrh = wrh_ref[...]
        wrs = wrs_ref[...]

        bidx = [g * NB + j for g in range(G)]
        mean_jT = [jnp.transpose(mean_scr[pl.ds(b, 1)].reshape(PLAN, A))
                   for b in bidx]
        std_jT = [jnp.transpose(std_scr[pl.ds(b, 1)].reshape(PLAN, A))
                  for b in bidx]
        h0 = [jnp.transpose(h0_ref[pl.ds(b, 1)].reshape(1, H))
              .astype(jnp.bfloat16) for b in bidx]
        s0 = [jnp.transpose(s0_ref[pl.ds(b, 1)].reshape(1, S))
              .astype(jnp.bfloat16) for b in bidx]

        def seg_bcast(cols, n):
            if len(cols) == 1:
                return jnp.broadcast_to(cols[0], (n, SEG))
            return jnp.concatenate(
                [jnp.broadcast_to(c, (n, SEG)) for c in cols], axis=1)

        GC = G // CH
        WC = GC * SEG
        cseg = [slice(c * WC, (c + 1) * WC) for c in range(CH)]
        cgrp = [slice(c * GC, (c + 1) * GC) for c in range(CH)]
        ret = [jnp.zeros((1, WC), jnp.float32) for _ in range(CH)]
        h = [None] * CH
        s = [None] * CH
        for p in range(PLAN):
            eps_p = eps_ref[0, j, p]
            a = [None] * CH
            for c in range(CH):
                m_full = seg_bcast([m[:, p:p + 1]
                                    for m in mean_jT[cgrp[c]]], A)
                sd_full = seg_bcast([sd[:, p:p + 1]
                                     for sd in std_jT[cgrp[c]]], A)
                a[c] = (m_full +
                        sd_full * eps_p[:, cseg[c]]).astype(jnp.bfloat16)
            for c in range(CH):
                if p == 0:
                    hh = seg_bcast([bdot(WhhT, col) for col in h0[cgrp[c]]], H)
                    ss = seg_bcast([bdot(WssT, col) for col in s0[cgrp[c]]], S)
                else:
                    hh = bdot(WhhT, h[c])
                    ss = bdot(WssT, s[c])
                h[c] = jnp.tanh(hh + bdot(WahT, a[c])).astype(jnp.bfloat16)
                s[c] = jnp.tanh(ss + bdot(WhsT, h[c])).astype(jnp.bfloat16)
                ret[c] = ret[c] + bdot(wrh, h[c]) + bdot(wrs, s[c])
        ret_full = ret[0] if CH == 1 else jnp.concatenate(ret, axis=1)
        ret_scr[pl.ds(j, 1)] = ret_full.reshape(1, 1, W)

    @pl.when(j == NB)
    def _select():
        ret = ret_scr[...].reshape(NB, W)
        bits = jax.lax.bitcast_convert_type(ret, jnp.int32)
        int_min = jnp.int32(_INT32_MIN)
        skey = jnp.where(bits >= 0, bits,
                         jnp.bitwise_xor(jnp.invert(bits), int_min))
        lane = jax.lax.broadcasted_iota(jnp.int32, (NB, W), 1)
        skey = jnp.where((lane & (SEG - 1)) >= CAND, int_min, skey)

        epsb = eps_ref[0]
        for g in range(G):
            sk = skey[:, g * SEG:(g + 1) * SEG]

            def count_ge(t):
                return jnp.sum((sk >= t).astype(jnp.int32), axis=1,
                               keepdims=True)

            zero = jnp.zeros((NB, 1), jnp.int32)
            t = jnp.where(count_ge(zero) >= TOPK, zero,
                          jnp.full((NB, 1), _INT32_MIN, jnp.int32))
            for bit in range(30, -1, -1):
                cand_t = t + jnp.int32(1 << bit)
                t = jnp.where(count_ge(cand_t) >= TOPK, cand_t, t)
            mask = sk >= t
            cnt = jnp.sum(mask.astype(jnp.float32), axis=1, keepdims=True)
            inv = (1.0 / cnt).reshape(NB, 1, 1)

            seg = epsb[:, :, :, g * SEG:(g + 1) * SEG]
            esel = jnp.where(mask[:, None, None, :], seg, 0.0)
            s1 = jnp.sum(esel, axis=3)
            s2 = jnp.sum(esel * esel, axis=3)
            mu = s1 * inv
            var = s2 * inv - mu * mu
            sd = jnp.sqrt(jnp.maximum(var, 0.0))
            rows = slice(g * NB, (g + 1) * NB)
            old_std = std_scr[rows]
            mean_scr[rows] = mean_scr[rows] + old_std * mu
            std_scr[rows] = old_std * sd

        @pl.when(it == ITERS - 1)
        def _out():
            out_ref[...] = mean_scr[...][:, 0, :]


@jax.jit
def kernel(hidden, state, W_hh, W_ah, W_ss, W_hs, w_rh, w_rs):
    base = jax.random.key(42)
    eps = jnp.stack([
        jax.random.normal(jax.random.fold_in(base, it), (PLAN, B, CAND, A),
                          dtype=hidden.dtype)
        for it in range(ITERS)
    ])
    epsT = jnp.transpose(eps, (0, 2, 1, 4, 3))
    epsT = jnp.pad(epsT, ((0, 0), (0, 0), (0, 0), (0, 0), (0, SEG - CAND)))
    epsT = epsT.reshape(ITERS, G, NB, PLAN, A, SEG)
    epsT = jnp.transpose(epsT, (0, 2, 3, 4, 1, 5)).reshape(
        ITERS, NB, PLAN, A, W)

    grid = (ITERS, NB + 1)
    out = pl.pallas_call(
        _body,
        grid=grid,
        in_specs=[
            pl.BlockSpec((1, NB, PLAN, A, W), lambda it, j: (it, 0, 0, 0, 0)),
            pl.BlockSpec((B, 1, H), lambda it, j: (0, 0, 0)),
            pl.BlockSpec((B, 1, S), lambda it, j: (0, 0, 0)),
            pl.BlockSpec((H, H), lambda it, j: (0, 0)),
            pl.BlockSpec((H, A), lambda it, j: (0, 0)),
            pl.BlockSpec((S, S), lambda it, j: (0, 0)),
            pl.BlockSpec((S, H), lambda it, j: (0, 0)),
            pl.BlockSpec((1, H), lambda it, j: (0, 0)),
            pl.BlockSpec((1, S), lambda it, j: (0, 0)),
        ],
        out_specs=pl.BlockSpec((B, A), lambda it, j: (0, 0)),
        out_shape=jax.ShapeDtypeStruct((B, A), jnp.float32),
        scratch_shapes=[
            pltpu.VMEM((NB, 1, W), jnp.float32),
            pltpu.VMEM((B, PLAN, A), jnp.float32),
            pltpu.VMEM((B, PLAN, A), jnp.float32),
        ],
    )(epsT, hidden.reshape(B, 1, H), state.reshape(B, 1, S),
      W_hh.T.astype(jnp.bfloat16), W_ah.T.astype(jnp.bfloat16),
      W_ss.T.astype(jnp.bfloat16), W_hs.T.astype(jnp.bfloat16),
      w_rh.reshape(1, H).astype(jnp.bfloat16),
      w_rs.reshape(1, S).astype(jnp.bfloat16))
    return out
